# baseline (speedup 1.0000x reference)
import functools

import jax
import jax.numpy as jnp
from jax import lax
from jax.experimental import pallas as pl
from jax.experimental.pallas import tpu as pltpu
from jax.experimental.pallas import tpu_sc as plsc

NC = 2
NS = 16
NW = NC * NS
CH = 128
LANES = 16


@functools.lru_cache(maxsize=None)
def _build_sc_kernel(n_rows: int, edim: int):
    assert n_rows % (NW * CH) == 0
    rows_per_w = n_rows // NW
    n_chunks = rows_per_w // CH
    assert n_chunks >= 4
    nb = n_rows // 50

    mesh = plsc.VectorSubcoreMesh(core_axis_name="c", subcore_axis_name="s")

    @functools.partial(
        pl.kernel,
        mesh=mesh,
        compiler_params=pltpu.CompilerParams(
            needs_layout_passes=False, use_tc_tiling_on_sc=True),
        out_type=(
            jax.ShapeDtypeStruct((n_rows,), jnp.float32),
            jax.ShapeDtypeStruct((n_rows,), jnp.float32),
            jax.ShapeDtypeStruct((nb * edim,), jnp.float32),
            jax.ShapeDtypeStruct((n_rows * edim,), jnp.float32),
            jax.ShapeDtypeStruct((n_rows * edim,), jnp.float32),
        ),
        scratch_types=[
            pltpu.VMEM((CH,), jnp.int32),
            pltpu.VMEM((CH,), jnp.int32),
            pltpu.VMEM((CH, 2 * edim), jnp.float32),
            pltpu.VMEM((CH,), jnp.int32),
            pltpu.VMEM((CH,), jnp.int32),
            pltpu.VMEM((CH,), jnp.int32),
            pltpu.VMEM((CH,), jnp.int32),
            pltpu.VMEM((CH,), jnp.int32),
            pltpu.VMEM((CH,), jnp.int32),
            pltpu.VMEM((CH,), jnp.int32),
            pltpu.VMEM((CH,), jnp.int32),
            pltpu.VMEM((CH, 2 * edim), jnp.float32),
            pltpu.VMEM((CH, 2 * edim), jnp.float32),
            pltpu.VMEM((CH, 2 * edim), jnp.float32),
            pltpu.VMEM((CH, 2 * edim), jnp.float32),
            pltpu.VMEM((CH * edim,), jnp.float32),
            pltpu.VMEM((CH * edim,), jnp.float32),
            pltpu.VMEM((CH * edim,), jnp.float32),
            pltpu.VMEM((CH * edim,), jnp.float32),
            pltpu.VMEM((CH,), jnp.float32),
            pltpu.VMEM((CH,), jnp.float32),
            pltpu.VMEM((CH,), jnp.float32),
            pltpu.VMEM((CH,), jnp.float32),
            pltpu.SemaphoreType.DMA,
            pltpu.SemaphoreType.DMA,
            pltpu.SemaphoreType.DMA,
            pltpu.SemaphoreType.DMA,
            pltpu.SemaphoreType.DMA,
            pltpu.SemaphoreType.DMA,
        ],
    )
    def sc_kernel(pos_hbm, neg_hbm, uid_hbm, upairs_hbm, ipairs_hbm,
                  plog_out, nlog_out, u_out, pos_out, neg_out,
                  uidx_v, upar_v, upr_v,
                  pidx0, pidx1, nidx0, nidx1,
                  pidx20, pidx21, nidx20, nidx21,
                  ppr0, ppr1, npr0, npr1,
                  ppk0, ppk1, npk0, npk1,
                  plog0, plog1, nlog0, nlog1,
                  semi0, semi1, semg0, semg1, semo0, semo1):
        wid = lax.axis_index("s") * NC + lax.axis_index("c")
        w0 = wid * rows_per_w
        lane_iota = lax.iota(jnp.int32, LANES)
        semi = (semi0, semi1)
        semg = (semg0, semg1)
        semo = (semo0, semo1)
        pidx_v = (pidx0, pidx1)
        nidx_v = (nidx0, nidx1)
        pidx2_v = (pidx20, pidx21)
        nidx2_v = (nidx20, nidx21)
        ppr_v = (ppr0, ppr1)
        npr_v = (npr0, npr1)
        ppk_v = (ppk0, ppk1)
        npk_v = (npk0, npk1)
        plog_v = (plog0, plog1)
        nlog_v = (nlog0, nlog1)

        def issue_idx(c, p):
            base = w0 + c * CH
            pltpu.async_copy(pos_hbm.at[pl.ds(base, CH)], pidx_v[p], semi[p])
            pltpu.async_copy(neg_hbm.at[pl.ds(base, CH)], nidx_v[p], semi[p])

        def wait_idx(p):
            pltpu.make_async_copy(
                pos_hbm.at[pl.ds(0, CH)], pidx_v[p], semi[p]).wait()
            pltpu.make_async_copy(
                neg_hbm.at[pl.ds(0, CH)], nidx_v[p], semi[p]).wait()

        def shift_idx(p):
            for g in range(CH // LANES):
                sl = pl.ds(g * LANES, LANES)
                pidx2_v[p][sl] = lax.shift_right_logical(pidx_v[p][sl], 1)
                nidx2_v[p][sl] = lax.shift_right_logical(nidx_v[p][sl], 1)

        def issue_gathers(p):
            pltpu.async_copy(ipairs_hbm.at[pidx2_v[p]], ppr_v[p], semg[p])
            pltpu.async_copy(ipairs_hbm.at[nidx2_v[p]], npr_v[p], semg[p])

        def wait_gathers(p):
            pltpu.make_async_copy(
                ipairs_hbm.at[pl.ds(0, CH)], ppr_v[p], semg[p]).wait()
            pltpu.make_async_copy(
                ipairs_hbm.at[pl.ds(0, CH)], npr_v[p], semg[p]).wait()

        def issue_rows_out(c, p):
            base = (w0 + c * CH) * edim
            pltpu.async_copy(
                ppk_v[p], pos_out.at[pl.ds(base, CH * edim)], semo[p])
            pltpu.async_copy(
                npk_v[p], neg_out.at[pl.ds(base, CH * edim)], semo[p])

        def issue_logits_out(c, p):
            base = w0 + c * CH
            pltpu.async_copy(plog_v[p], plog_out.at[pl.ds(base, CH)], semo[p])
            pltpu.async_copy(nlog_v[p], nlog_out.at[pl.ds(base, CH)], semo[p])

        def wait_outs(p):
            pltpu.make_async_copy(
                ppk_v[p], pos_out.at[pl.ds(0, CH * edim)], semo[p]).wait()
            pltpu.make_async_copy(
                npk_v[p], neg_out.at[pl.ds(0, CH * edim)], semo[p]).wait()
            pltpu.make_async_copy(
                plog_v[p], plog_out.at[pl.ds(0, CH)], semo[p]).wait()
            pltpu.make_async_copy(
                nlog_v[p], nlog_out.at[pl.ds(0, CH)], semo[p]).wait()

        def compute(c, p):
            def group_body(g, _):
                r0 = g * LANES
                rowv = lane_iota + r0
                bl = lax.div(c * CH + r0 + lane_iota, 50)
                ppar = (pidx_v[p][pl.ds(r0, LANES)] & 1) * edim
                npar = (nidx_v[p][pl.ds(r0, LANES)] & 1) * edim
                uq = plsc.load_gather(upar_v, [bl])
                pacc = jnp.zeros((LANES,), jnp.float32)
                nacc = jnp.zeros((LANES,), jnp.float32)
                flat = rowv * edim
                for cc in range(edim):
                    hcol = plsc.load_gather(upr_v, [bl, uq + cc])
                    pvv = plsc.load_gather(ppr_v[p], [rowv, ppar + cc])
                    nvv = plsc.load_gather(npr_v[p], [rowv, npar + cc])
                    pacc = pacc + hcol * pvv
                    nacc = nacc + hcol * nvv
                    plsc.store_scatter(ppk_v[p], [flat + cc], pvv)
                    plsc.store_scatter(npk_v[p], [flat + cc], nvv)
                plog_v[p][pl.ds(r0, LANES)] = pacc
                nlog_v[p][pl.ds(r0, LANES)] = nacc
                return 0

            lax.fori_loop(0, CH // LANES, group_body, 0)

        def step(c, par, wait_out, issue_next, issue_idx2):
            q = 1 - par
            if issue_next:
                wait_idx(q)
                shift_idx(q)
                if wait_out:
                    wait_outs(q)
                issue_gathers(q)
            wait_gathers(par)
            compute(c, par)
            issue_rows_out(c, par)
            issue_logits_out(c, par)
            if issue_idx2:
                issue_idx(c + 2, par)

        pltpu.sync_copy(uid_hbm.at[pl.ds(wid * CH, CH)], uidx_v)
        for g in range(CH // LANES):
            sl = pl.ds(g * LANES, LANES)
            upar_v[sl] = (uidx_v[sl] & 1) * edim
            uidx_v[sl] = lax.shift_right_logical(uidx_v[sl], 1)
        pltpu.async_copy(upairs_hbm.at[uidx_v], upr_v, semg0).wait()
        for g in range(CH // LANES):
            rowv = lane_iota + g * LANES
            uq = upar_v[pl.ds(g * LANES, LANES)]
            flat = rowv * edim
            for cc in range(edim):
                val = plsc.load_gather(upr_v, [rowv, uq + cc])
                plsc.store_scatter(ppk0, [flat + cc], val)
        pltpu.sync_copy(ppk0, u_out.at[pl.ds(wid * CH * edim, CH * edim)])

        issue_idx(0, 0)
        issue_idx(1, 1)
        wait_idx(0)
        shift_idx(0)
        issue_gathers(0)

        step(0, 0, wait_out=False, issue_next=True, issue_idx2=True)
        step(1, 1, wait_out=True, issue_next=True, issue_idx2=True)

        def pair_body(j, _):
            c0 = 2 * j
            step(c0, 0, wait_out=True, issue_next=True, issue_idx2=True)
            step(c0 + 1, 1, wait_out=True, issue_next=True, issue_idx2=True)
            return 0

        lax.fori_loop(1, n_chunks // 2 - 1, pair_body, 0)

        step(n_chunks - 2, 0, wait_out=True, issue_next=True, issue_idx2=False)
        step(n_chunks - 1, 1, wait_out=True, issue_next=False, issue_idx2=False)

        wait_outs(0)
        wait_outs(1)

    return sc_kernel


def kernel(uid, seq, pos, neg, nbr, nbr_iid, user_embs, item_embs):
    b, l = pos.shape
    edim = user_embs.shape[1]
    n_rows = b * l
    upairs = user_embs.reshape(user_embs.shape[0] // 2, 2 * edim)
    ipairs = item_embs.reshape(item_embs.shape[0] // 2, 2 * edim)
    sc = _build_sc_kernel(n_rows, edim)
    plog, nlog, u_flat, pos_flat, neg_flat = sc(
        pos.reshape(-1), neg.reshape(-1), uid, upairs, ipairs)
    u_rows = u_flat.reshape(b, edim)
    hu = jnp.broadcast_to(u_rows[:, None, :], (b, l, edim))
    return (plog.reshape(b, l), nlog.reshape(b, l),
            hu, pos_flat.reshape(b, l, edim),
            neg_flat.reshape(b, l, edim))


# speedup vs baseline: 1.9387x; 1.9387x over previous
import functools

import jax
import jax.numpy as jnp
from jax import lax
from jax.experimental import pallas as pl
from jax.experimental.pallas import tpu as pltpu
from jax.experimental.pallas import tpu_sc as plsc

NC = 2
NS = 16
NW = NC * NS
CH = 128
LANES = 16


@functools.lru_cache(maxsize=None)
def _build_user_kernel(nb: int, edim: int):
    assert nb % NW == 0
    per_w = nb // NW
    mesh = plsc.VectorSubcoreMesh(core_axis_name="c", subcore_axis_name="s")

    @functools.partial(
        pl.kernel,
        mesh=mesh,
        compiler_params=pltpu.CompilerParams(
            needs_layout_passes=False, use_tc_tiling_on_sc=False),
        out_type=jax.ShapeDtypeStruct((nb, edim), jnp.float32),
        scratch_types=[
            pltpu.VMEM((per_w,), jnp.int32),
            pltpu.VMEM((per_w, edim), jnp.float32),
            pltpu.SemaphoreType.DMA,
        ],
    )
    def user_kernel(uid_hbm, uembs_hbm, u_out, uidx_v, usel_v, sem):
        wid = lax.axis_index("s") * NC + lax.axis_index("c")
        pltpu.sync_copy(uid_hbm.at[pl.ds(wid * per_w, per_w)], uidx_v)
        pltpu.async_copy(uembs_hbm.at[uidx_v], usel_v, sem).wait()
        pltpu.sync_copy(usel_v, u_out.at[pl.ds(wid * per_w, per_w)])

    return user_kernel


@functools.lru_cache(maxsize=None)
def _build_sc_kernel(n_rows: int, edim: int):
    assert n_rows % (NW * CH) == 0
    rows_per_w = n_rows // NW
    n_chunks = rows_per_w // CH
    assert n_chunks >= 4
    q4 = edim // LANES

    mesh = plsc.VectorSubcoreMesh(core_axis_name="c", subcore_axis_name="s")

    @functools.partial(
        pl.kernel,
        mesh=mesh,
        compiler_params=pltpu.CompilerParams(
            needs_layout_passes=False, use_tc_tiling_on_sc=False),
        out_type=(
            jax.ShapeDtypeStruct((n_rows,), jnp.float32),
            jax.ShapeDtypeStruct((n_rows,), jnp.float32),
            jax.ShapeDtypeStruct((n_rows, edim), jnp.float32),
            jax.ShapeDtypeStruct((n_rows, edim), jnp.float32),
        ),
        scratch_types=[
            pltpu.VMEM((CH,), jnp.int32),
            pltpu.VMEM((CH, edim), jnp.float32),
            pltpu.VMEM((2, CH), jnp.int32),
            pltpu.VMEM((2, CH), jnp.int32),
            pltpu.VMEM((2, CH, edim), jnp.float32),
            pltpu.VMEM((2, CH, edim), jnp.float32),
            pltpu.VMEM((2, CH), jnp.float32),
            pltpu.VMEM((2, CH), jnp.float32),
            pltpu.SemaphoreType.DMA,
            pltpu.SemaphoreType.DMA,
            pltpu.SemaphoreType.DMA,
            pltpu.SemaphoreType.DMA,
            pltpu.SemaphoreType.DMA,
            pltpu.SemaphoreType.DMA,
        ],
    )
    def sc_kernel(pos_hbm, neg_hbm, uid_hbm, uembs_hbm, iembs_hbm,
                  plog_out, nlog_out, pos_out, neg_out,
                  uidx_v, usel_v, pidx_v, nidx_v, pos_v, neg_v,
                  plog_v, nlog_v, semi0, semi1, semg0, semg1, semo0, semo1):
        wid = lax.axis_index("s") * NC + lax.axis_index("c")
        w0 = wid * rows_per_w
        lane_iota = lax.iota(jnp.int32, LANES)
        semi = (semi0, semi1)
        semg = (semg0, semg1)
        semo = (semo0, semo1)
        idx_srcs = (pos_hbm, neg_hbm)

        def idx_refs(p):
            return (pidx_v.at[p], nidx_v.at[p])

        def row_refs(p):
            return (pos_v.at[p], neg_v.at[p])

        def row_outs():
            return (pos_out, neg_out)

        def issue_idx(c, p):
            base = w0 + c * CH
            for src, dst in zip(idx_srcs, idx_refs(p)):
                pltpu.async_copy(src.at[pl.ds(base, CH)], dst, semi[p])

        def wait_idx(p):
            for src, dst in zip(idx_srcs, idx_refs(p)):
                pltpu.make_async_copy(src.at[pl.ds(0, CH)], dst, semi[p]).wait()

        def issue_gathers(p):
            pltpu.async_copy(iembs_hbm.at[pidx_v.at[p]], pos_v.at[p], semg[p])
            pltpu.async_copy(iembs_hbm.at[nidx_v.at[p]], neg_v.at[p], semg[p])

        def wait_gathers(p):
            for dst in row_refs(p):
                pltpu.make_async_copy(
                    iembs_hbm.at[pl.ds(0, CH)], dst, semg[p]).wait()

        def issue_rows_out(c, p):
            base = w0 + c * CH
            for src, out in zip(row_refs(p), row_outs()):
                pltpu.async_copy(src, out.at[pl.ds(base, CH)], semo[p])

        def issue_logits_out(c, p):
            base = w0 + c * CH
            pltpu.async_copy(plog_v.at[p], plog_out.at[pl.ds(base, CH)], semo[p])
            pltpu.async_copy(nlog_v.at[p], nlog_out.at[pl.ds(base, CH)], semo[p])

        def wait_outs(p):
            for src, out in zip(row_refs(p), row_outs()):
                pltpu.make_async_copy(src, out.at[pl.ds(0, CH)], semo[p]).wait()
            pltpu.make_async_copy(
                plog_v.at[p], plog_out.at[pl.ds(0, CH)], semo[p]).wait()
            pltpu.make_async_copy(
                nlog_v.at[p], nlog_out.at[pl.ds(0, CH)], semo[p]).wait()

        def compute(c, p):
            def group_body(g, _):
                r0 = g * LANES
                rbase = c * CH + r0
                psum = jnp.zeros((LANES,), jnp.float32)
                nsum = jnp.zeros((LANES,), jnp.float32)
                for i in range(LANES):
                    r = r0 + i
                    bl = lax.div(rbase + i, 50)
                    pacc = None
                    nacc = None
                    for q in range(q4):
                        h = usel_v[bl, pl.ds(q * LANES, LANES)]
                        pv = pos_v[p, r, pl.ds(q * LANES, LANES)]
                        ng = neg_v[p, r, pl.ds(q * LANES, LANES)]
                        pacc = h * pv if pacc is None else pacc + h * pv
                        nacc = h * ng if nacc is None else nacc + h * ng
                    lane_is_i = lane_iota == i
                    psum = jnp.where(lane_is_i, jnp.sum(pacc), psum)
                    nsum = jnp.where(lane_is_i, jnp.sum(nacc), nsum)
                plog_v[p, pl.ds(r0, LANES)] = psum
                nlog_v[p, pl.ds(r0, LANES)] = nsum
                return 0

            lax.fori_loop(0, CH // LANES, group_body, 0)

        def step(c, par, wait_out, issue_next, issue_idx2):
            q = 1 - par
            if issue_next:
                wait_idx(q)
                if wait_out:
                    wait_outs(q)
                issue_gathers(q)
            wait_gathers(par)
            if issue_idx2:
                issue_idx(c + 2, par)
            issue_rows_out(c, par)
            compute(c, par)
            issue_logits_out(c, par)

        pltpu.sync_copy(uid_hbm.at[pl.ds(wid * CH, CH)], uidx_v)
        pltpu.async_copy(uembs_hbm.at[uidx_v], usel_v, semg0).wait()

        issue_idx(0, 0)
        issue_idx(1, 1)
        wait_idx(0)
        issue_gathers(0)

        step(0, 0, wait_out=False, issue_next=True, issue_idx2=True)
        step(1, 1, wait_out=True, issue_next=True, issue_idx2=True)

        def pair_body(j, _):
            c0 = 2 * j
            step(c0, 0, wait_out=True, issue_next=True, issue_idx2=True)
            step(c0 + 1, 1, wait_out=True, issue_next=True, issue_idx2=True)
            return 0

        lax.fori_loop(1, n_chunks // 2 - 1, pair_body, 0)

        step(n_chunks - 2, 0, wait_out=True, issue_next=True, issue_idx2=False)
        step(n_chunks - 1, 1, wait_out=True, issue_next=False, issue_idx2=False)

        wait_outs(0)
        wait_outs(1)

    return sc_kernel


def kernel(uid, seq, pos, neg, nbr, nbr_iid, user_embs, item_embs):
    b, l = pos.shape
    edim = user_embs.shape[1]
    n_rows = b * l
    uk = _build_user_kernel(b, edim)
    u_rows = uk(uid, user_embs)
    sc = _build_sc_kernel(n_rows, edim)
    plog, nlog, pos_hi, neg_hi = sc(
        pos.reshape(-1), neg.reshape(-1), uid, user_embs, item_embs)
    hu = jnp.broadcast_to(u_rows[:, None, :], (b, l, edim))
    return (plog.reshape(b, l), nlog.reshape(b, l),
            hu, pos_hi.reshape(b, l, edim),
            neg_hi.reshape(b, l, edim))
